# baseline (speedup 1.0000x reference)
import functools

import jax
import jax.numpy as jnp
from jax import lax
from jax.experimental import pallas as pl
from jax.experimental.pallas import tpu as pltpu
from jax.experimental.pallas import tpu_sc as plsc

B = 16384
TOPK = 1024
NSUB = 16
CHUNK = B // NSUB
NV = CHUNK // 16
UN = 8
KF = float(TOPK)
ROUNDS = ((24, 128), (18, 64), (12, 64), (6, 64), (0, 64))
NR = len(ROUNDS)


def _make_sc_kernel():
    mesh = plsc.VectorSubcoreMesh(core_axis_name="c", subcore_axis_name="s",
                                  num_cores=1)

    @functools.partial(
        pl.kernel,
        mesh=mesh,
        out_type=jax.ShapeDtypeStruct((16,), jnp.float32),
        compiler_params=pltpu.CompilerParams(needs_layout_passes=False),
        scratch_types=[
            pltpu.VMEM((CHUNK,), jnp.float32),
            pltpu.VMEM((CHUNK,), jnp.float32),
            pltpu.VMEM((CHUNK,), jnp.float32),
            pltpu.VMEM((CHUNK,), jnp.int32),
            pltpu.VMEM((16, 16), jnp.float32),
            pltpu.VMEM((16, 16), jnp.float32),
            pltpu.VMEM((16, 16), jnp.float32),
            pltpu.VMEM((16,), jnp.int32),
            pltpu.VMEM((16,), jnp.float32),
            pltpu.VMEM((NSUB, 16, 16), jnp.float32),
        ] + [pltpu.VMEM_SHARED((NSUB, 16, 16), jnp.float32) for _ in range(NR)],
    )
    def topk_sum_kernel(p_hbm, t_hbm, s_hbm, out_hbm,
                        p_v, t_v, s_v, bits_v, hist_v, gat_v, zero_v,
                        idx_v, res_v, gat3_v, *shared):
        c = lax.axis_index("c")
        w = lax.axis_index("s")
        base = w * CHUNK
        pltpu.sync_copy(p_hbm.at[pl.ds(base, CHUNK)], p_v)
        pltpu.sync_copy(t_hbm.at[pl.ds(base, CHUNK)], t_v)
        pltpu.sync_copy(s_hbm.at[pl.ds(base, CHUNK)], s_v)

        lane = lax.iota(jnp.int32, 16)
        zf16 = jnp.zeros((16,), jnp.float32)
        one16f = zf16 + 1.0
        idx_v[...] = lane
        for r in range(16):
            zero_v[r, :] = zf16

        gdn = lax.GatherDimensionNumbers(
            offset_dims=(), collapsed_slice_dims=(0,), start_index_map=(0,))

        def shuffle(x, idx):
            return lax.gather(
                x, idx[:, None], gdn, (1,),
                mode=lax.GatherScatterMode.PROMISE_IN_BOUNDS)

        def vsum(x):
            for sh in (8, 4, 2, 1):
                x = x + shuffle(x, lane ^ sh)
            return x

        def bcast0(x):
            return shuffle(x, jnp.zeros((16,), jnp.int32))

        def compute_bits(i8, carry):
            for j in range(UN):
                sl = pl.ds((i8 * UN + j) * 16, 16)
                d = p_v[sl] - s_v[sl]
                l = t_v[sl] * d * d
                bits_v[sl] = lax.bitcast_convert_type(l, jnp.int32)
            return carry

        lax.fori_loop(0, NV // UN, compute_bits, 0)

        P = jnp.zeros((16,), jnp.int32)
        C_gt = zf16
        V_gt = zf16

        for rnd, (shift, nbins) in enumerate(ROUNDS):
            nb = nbins // 16
            first = rnd == 0

            for r in range(16):
                hist_v[r, :] = zf16

            span = jnp.int32(nbins << shift) if not first else None

            def sbody(i8, carry):
                for j in range(UN):
                    v = bits_v[pl.ds((i8 * UN + j) * 16, 16)]
                    diff = v - P
                    if first:
                        cand = v >= P
                    else:
                        cand = (v >= P) & (diff < span)
                    digit = jnp.right_shift(diff, shift) & (nbins - 1)
                    crow = jnp.right_shift(digit, 4)
                    ccol = digit & 15
                    lval = lax.bitcast_convert_type(v, jnp.float32)
                    plsc.addupdate_scatter(hist_v, [crow, ccol], one16f,
                                           mask=cand)
                    plsc.addupdate_scatter(hist_v, [crow + nb, ccol], lval,
                                           mask=cand)
                return carry

            lax.fori_loop(0, NV // UN, sbody, 0)

            pltpu.sync_copy(hist_v, shared[rnd].at[w])
            plsc.subcore_barrier()
            pltpu.sync_copy(shared[rnd], gat3_v)

            cnt = []
            val = []
            for bv in range(nb):
                a = gat3_v[0, bv, :]
                b2 = gat3_v[0, nb + bv, :]
                for r in range(1, NSUB):
                    a = a + gat3_v[r, bv, :]
                    b2 = b2 + gat3_v[r, nb + bv, :]
                cnt.append(a)
                val.append(b2)

            suf = [None] * nb
            carry = zf16
            for bv in range(nb - 1, -1, -1):
                x = cnt[bv]
                for sh in (1, 2, 4, 8):
                    x = x + jnp.where(lane < 16 - sh,
                                      shuffle(x, (lane + sh) & 15), 0.0)
                x = x + carry
                carry = bcast0(x)
                suf[bv] = x

            R = KF - C_gt
            acc_d = zf16
            for bv in range(nb):
                bin_id = lane + bv * 16
                acc_d = acc_d + jnp.where((suf[bv] >= R) & (bin_id >= 1),
                                          1.0, 0.0)
            dstar = vsum(acc_d)
            dstar_i = dstar.astype(jnp.int32)

            acc_s = zf16
            acc_v = zf16
            dnext = dstar + 1.0
            for bv in range(nb):
                bin_id = (lane + bv * 16).astype(jnp.float32)
                acc_s = acc_s + jnp.where(bin_id == dnext, suf[bv], 0.0)
                acc_v = acc_v + jnp.where(bin_id >= dnext, val[bv], 0.0)
            snext = vsum(acc_s)
            vnext = vsum(acc_v)

            P = P + lax.shift_left(dstar_i, shift)
            C_gt = C_gt + snext
            V_gt = V_gt + vnext

        tf = lax.bitcast_convert_type(P, jnp.float32)
        res_v[...] = (V_gt + (KF - C_gt) * tf) * (1.0 / KF)

        @pl.when(jnp.logical_and(c == 0, w == 0))
        def _():
            pltpu.sync_copy(res_v, out_hbm)

    return topk_sum_kernel


_sc_kernel = _make_sc_kernel()


def kernel(inputs, targets, k):
    del k
    out = _sc_kernel(inputs.reshape(B), targets[:, 0], targets[:, 1])
    return out[0]


# speedup vs baseline: 1.2501x; 1.2501x over previous
import functools

import jax
import jax.numpy as jnp
from jax import lax
from jax.experimental import pallas as pl
from jax.experimental.pallas import tpu as pltpu
from jax.experimental.pallas import tpu_sc as plsc

B = 16384
TOPK = 1024
NSUB = 16
CHUNK = B // NSUB
NV = CHUNK // 16
KF = float(TOPK)
NBINS = 128
HB = 2 * NBINS
ROUNDS = ((24, 128, True), (18, 64, False), (12, 64, False),
          (6, 64, False), (0, 64, False))


def _make_sc_kernel():
    mesh = plsc.VectorSubcoreMesh(core_axis_name="c", subcore_axis_name="s",
                                  num_cores=1)

    @functools.partial(
        pl.kernel,
        mesh=mesh,
        out_type=jax.ShapeDtypeStruct((16,), jnp.float32),
        compiler_params=pltpu.CompilerParams(needs_layout_passes=False),
        scratch_types=[
            pltpu.VMEM((CHUNK,), jnp.float32),
            pltpu.VMEM((CHUNK,), jnp.float32),
            pltpu.VMEM((CHUNK,), jnp.float32),
            pltpu.VMEM((CHUNK,), jnp.int32),
            pltpu.VMEM((HB,), jnp.float32),
            pltpu.VMEM((NSUB, HB), jnp.float32),
            pltpu.VMEM((16,), jnp.float32),
            pltpu.VMEM_SHARED((2, NSUB, HB), jnp.float32),
        ],
    )
    def topk_sum_kernel(p_hbm, t_hbm, s_hbm, out_hbm,
                        p_v, t_v, s_v, bits_v, hist_v, gat_v, res_v, shared):
        c = lax.axis_index("c")
        w = lax.axis_index("s")
        base = w * CHUNK
        pltpu.sync_copy(p_hbm.at[pl.ds(base, CHUNK)], p_v)
        pltpu.sync_copy(t_hbm.at[pl.ds(base, CHUNK)], t_v)
        pltpu.sync_copy(s_hbm.at[pl.ds(base, CHUNK)], s_v)

        lane = lax.iota(jnp.int32, 16)
        zero16 = jnp.zeros((16,), jnp.int32)
        one16f = jnp.zeros((16,), jnp.float32) + 1.0

        gdn = lax.GatherDimensionNumbers(
            offset_dims=(), collapsed_slice_dims=(0,), start_index_map=(0,))

        def shuffle(x, idx):
            return lax.gather(
                x, idx[:, None], gdn, (1,),
                mode=lax.GatherScatterMode.PROMISE_IN_BOUNDS)

        def vsum(x):
            for sh in (8, 4, 2, 1):
                x = x + shuffle(x, lane ^ sh)
            return x

        def bcast0(x):
            return shuffle(x, zero16)

        def compute_bits(i, carry):
            sl = pl.ds(i * 16, 16)
            d = p_v[sl] - s_v[sl]
            l = t_v[sl] * d * d
            bits_v[sl] = lax.bitcast_convert_type(l, jnp.int32)
            return carry

        lax.fori_loop(0, NV, compute_bits, 0)

        P = zero16
        C_gt = jnp.zeros((16,), jnp.float32)
        V_gt = jnp.zeros((16,), jnp.float32)

        for rnd, (shift, nbins, first) in enumerate(ROUNDS):
            parity = rnd % 2
            nb = nbins // 16

            def zbody(i, carry):
                hist_v[pl.ds(i * 16, 16)] = jnp.zeros((16,), jnp.float32)
                return carry

            lax.fori_loop(0, HB // 16, zbody, 0)

            span = jnp.int32(nbins << shift) if not first else None

            def sbody(i, carry):
                v = bits_v[pl.ds(i * 16, 16)]
                diff = v - P
                if first:
                    cand = v >= P
                else:
                    cand = (v >= P) & (diff < span)
                digit = jnp.right_shift(diff, shift) & (nbins - 1)
                lval = lax.bitcast_convert_type(v, jnp.float32)
                plsc.addupdate_scatter(hist_v, [digit], one16f, mask=cand)
                plsc.addupdate_scatter(hist_v, [digit + NBINS], lval,
                                       mask=cand)
                return carry

            lax.fori_loop(0, NV, sbody, 0)

            pltpu.sync_copy(hist_v, shared.at[parity, w])
            plsc.subcore_barrier()
            pltpu.sync_copy(shared.at[parity], gat_v)

            def mbody(r, accs):
                new = []
                for bv in range(nb):
                    new.append(accs[bv] + gat_v[r, pl.ds(bv * 16, 16)])
                for bv in range(nb):
                    new.append(accs[nb + bv]
                               + gat_v[r, pl.ds(NBINS + bv * 16, 16)])
                return tuple(new)

            accs = lax.fori_loop(
                0, NSUB, mbody,
                tuple(jnp.zeros((16,), jnp.float32) for _ in range(2 * nb)))
            cnt = accs[:nb]
            val = accs[nb:]

            suf = [None] * nb
            carry = jnp.zeros((16,), jnp.float32)
            for bv in range(nb - 1, -1, -1):
                x = cnt[bv]
                for sh in (1, 2, 4, 8):
                    x = x + jnp.where(lane < 16 - sh,
                                      shuffle(x, (lane + sh) & 15), 0.0)
                x = x + carry
                carry = bcast0(x)
                suf[bv] = x

            R = KF - C_gt
            acc_d = jnp.zeros((16,), jnp.float32)
            for bv in range(nb):
                bin_id = lane + bv * 16
                acc_d = acc_d + jnp.where((suf[bv] >= R) & (bin_id >= 1),
                                          1.0, 0.0)
            dstar = vsum(acc_d)
            dstar_i = dstar.astype(jnp.int32)

            acc_s = jnp.zeros((16,), jnp.float32)
            acc_v = jnp.zeros((16,), jnp.float32)
            dnext = dstar + 1.0
            for bv in range(nb):
                bin_id = (lane + bv * 16).astype(jnp.float32)
                acc_s = acc_s + jnp.where(bin_id == dnext, suf[bv], 0.0)
                acc_v = acc_v + jnp.where(bin_id >= dnext, val[bv], 0.0)
            snext = vsum(acc_s)
            vnext = vsum(acc_v)

            P = P + lax.shift_left(dstar_i, shift)
            C_gt = C_gt + snext
            V_gt = V_gt + vnext

        tf = lax.bitcast_convert_type(P, jnp.float32)
        res_v[...] = (V_gt + (KF - C_gt) * tf) * (1.0 / KF)

        @pl.when(jnp.logical_and(c == 0, w == 0))
        def _():
            pltpu.sync_copy(res_v, out_hbm)

    return topk_sum_kernel


_sc_kernel = _make_sc_kernel()


def kernel(inputs, targets, k):
    del k
    out = _sc_kernel(inputs.reshape(B), targets[:, 0], targets[:, 1])
    return out[0]


# speedup vs baseline: 1.2725x; 1.0179x over previous
import functools

import jax
import jax.numpy as jnp
from jax import lax
from jax.experimental import pallas as pl
from jax.experimental.pallas import tpu as pltpu
from jax.experimental.pallas import tpu_sc as plsc

B = 16384
TOPK = 1024
NSUB = 16
CHUNK = B // NSUB
NV = CHUNK // 16
KF = float(TOPK)
NBINS = 128
HB = 2 * NBINS
ROUNDS = ((24, 128, True), (18, 64, False), (12, 64, False),
          (6, 64, False), (0, 64, False))


def _make_sc_kernel():
    mesh = plsc.VectorSubcoreMesh(core_axis_name="c", subcore_axis_name="s",
                                  num_cores=1)

    @functools.partial(
        pl.kernel,
        mesh=mesh,
        out_type=jax.ShapeDtypeStruct((16,), jnp.float32),
        compiler_params=pltpu.CompilerParams(needs_layout_passes=False),
        scratch_types=[
            pltpu.VMEM((CHUNK,), jnp.float32),
            pltpu.VMEM((CHUNK,), jnp.float32),
            pltpu.VMEM((CHUNK,), jnp.float32),
            pltpu.VMEM((CHUNK,), jnp.int32),
            pltpu.VMEM((HB,), jnp.float32),
            pltpu.VMEM((NSUB * HB,), jnp.float32),
            pltpu.VMEM((16,), jnp.float32),
            pltpu.VMEM_SHARED((2, NSUB * HB), jnp.float32),
        ],
    )
    def topk_sum_kernel(p_hbm, t_hbm, s_hbm, out_hbm,
                        p_v, t_v, s_v, bits_v, hist_v, gat_v, res_v, shared):
        c = lax.axis_index("c")
        w = lax.axis_index("s")
        base = w * CHUNK
        pltpu.sync_copy(p_hbm.at[pl.ds(base, CHUNK)], p_v)
        pltpu.sync_copy(t_hbm.at[pl.ds(base, CHUNK)], t_v)
        pltpu.sync_copy(s_hbm.at[pl.ds(base, CHUNK)], s_v)

        lane = lax.iota(jnp.int32, 16)
        zero16 = jnp.zeros((16,), jnp.int32)
        one16f = jnp.zeros((16,), jnp.float32) + 1.0

        gdn = lax.GatherDimensionNumbers(
            offset_dims=(), collapsed_slice_dims=(0,), start_index_map=(0,))

        def shuffle(x, idx):
            return lax.gather(
                x, idx[:, None], gdn, (1,),
                mode=lax.GatherScatterMode.PROMISE_IN_BOUNDS)

        def vsum(x):
            for sh in (8, 4, 2, 1):
                x = x + shuffle(x, lane ^ sh)
            return x

        def bcast0(x):
            return shuffle(x, zero16)

        def compute_bits(i, carry):
            sl = pl.ds(i * 16, 16)
            d = p_v[sl] - s_v[sl]
            l = t_v[sl] * d * d
            bits_v[sl] = lax.bitcast_convert_type(l, jnp.int32)
            return carry

        lax.fori_loop(0, NV, compute_bits, 0)

        P = zero16
        C_gt = jnp.zeros((16,), jnp.float32)
        V_gt = jnp.zeros((16,), jnp.float32)

        for rnd, (shift, nbins, first) in enumerate(ROUNDS):
            parity = rnd % 2
            nb = nbins // 16

            hw = 2 * nbins

            def zbody(i, carry):
                hist_v[pl.ds(i * 16, 16)] = jnp.zeros((16,), jnp.float32)
                return carry

            lax.fori_loop(0, hw // 16, zbody, 0)

            span = jnp.int32(nbins << shift) if not first else None

            def sbody(i, carry):
                v = bits_v[pl.ds(i * 16, 16)]
                diff = v - P
                if first:
                    cand = v >= P
                else:
                    cand = (v >= P) & (diff < span)
                digit = jnp.right_shift(diff, shift) & (nbins - 1)
                lval = lax.bitcast_convert_type(v, jnp.float32)
                plsc.addupdate_scatter(hist_v, [digit], one16f, mask=cand)
                plsc.addupdate_scatter(hist_v, [digit + nbins], lval,
                                       mask=cand)
                return carry

            lax.fori_loop(0, NV, sbody, 0)

            pltpu.sync_copy(hist_v.at[pl.ds(0, hw)],
                            shared.at[parity, pl.ds(w * hw, hw)])
            plsc.subcore_barrier()
            pltpu.sync_copy(shared.at[parity, pl.ds(0, NSUB * hw)],
                            gat_v.at[pl.ds(0, NSUB * hw)])

            def mbody(r, accs):
                off = r * hw
                new = []
                for bv in range(nb):
                    new.append(accs[bv] + gat_v[pl.ds(off + bv * 16, 16)])
                for bv in range(nb):
                    new.append(accs[nb + bv]
                               + gat_v[pl.ds(off + nbins + bv * 16, 16)])
                return tuple(new)

            accs = lax.fori_loop(
                0, NSUB, mbody,
                tuple(jnp.zeros((16,), jnp.float32) for _ in range(2 * nb)))
            cnt = accs[:nb]
            val = accs[nb:]

            suf = [None] * nb
            carry = jnp.zeros((16,), jnp.float32)
            for bv in range(nb - 1, -1, -1):
                x = cnt[bv]
                for sh in (1, 2, 4, 8):
                    x = x + jnp.where(lane < 16 - sh,
                                      shuffle(x, (lane + sh) & 15), 0.0)
                x = x + carry
                carry = bcast0(x)
                suf[bv] = x

            R = KF - C_gt
            acc_d = jnp.zeros((16,), jnp.float32)
            for bv in range(nb):
                bin_id = lane + bv * 16
                acc_d = acc_d + jnp.where((suf[bv] >= R) & (bin_id >= 1),
                                          1.0, 0.0)
            dstar = vsum(acc_d)
            dstar_i = dstar.astype(jnp.int32)

            acc_s = jnp.zeros((16,), jnp.float32)
            acc_v = jnp.zeros((16,), jnp.float32)
            dnext = dstar + 1.0
            for bv in range(nb):
                bin_id = (lane + bv * 16).astype(jnp.float32)
                acc_s = acc_s + jnp.where(bin_id == dnext, suf[bv], 0.0)
                acc_v = acc_v + jnp.where(bin_id >= dnext, val[bv], 0.0)
            snext = vsum(acc_s)
            vnext = vsum(acc_v)

            P = P + lax.shift_left(dstar_i, shift)
            C_gt = C_gt + snext
            V_gt = V_gt + vnext

        tf = lax.bitcast_convert_type(P, jnp.float32)
        res_v[...] = (V_gt + (KF - C_gt) * tf) * (1.0 / KF)

        @pl.when(jnp.logical_and(c == 0, w == 0))
        def _():
            pltpu.sync_copy(res_v, out_hbm)

    return topk_sum_kernel


_sc_kernel = _make_sc_kernel()


def kernel(inputs, targets, k):
    del k
    out = _sc_kernel(inputs.reshape(B), targets[:, 0], targets[:, 1])
    return out[0]
